# baseline (speedup 1.0000x reference)
import jax
import jax.numpy as jnp
from jax import lax
from jax.experimental import pallas as pl
from jax.experimental.pallas import tpu as pltpu

_B, _C, _N, _T = 4, 64, 1024, 12
_K = int(_N * 0.8)
_HI = lax.Precision.HIGHEST
_RB = 16


def _dgcn_kernel(xt_ref, Wc1_ref, b1_ref, WgT_ref, a_ref, aP_ref, memT_ref,
                 cwa_ref, cw_ref, Wc2_ref, b2_ref, emb_ref, tri_ref, out_ref,
                 a1s, a2s, gts, eqs, adjs, zms):
    C, N, T, K, RB = _C, _N, _T, _K, _RB
    bf = jnp.bfloat16
    f32 = jnp.float32
    one_b = jnp.bfloat16(1.0)
    xt = xt_ref[0]
    x2 = xt.reshape(C, T * N)

    x1 = jnp.dot(Wc1_ref[...].astype(bf), x2.astype(bf),
                 preferred_element_type=f32)
    x1 = (x1 + b1_ref[...]).astype(bf)

    xs = (jnp.dot(Wc1_ref[...], xt.reshape(C, T, N).sum(axis=1),
                  precision=_HI) + jnp.float32(T) * b1_ref[...])
    hT = jnp.dot(WgT_ref[...], xs, precision=_HI)
    h_ = hT.T

    a1s[...] = jnp.dot(h_, memT_ref[...], precision=_HI) * 0.125
    a2s[...] = jnp.dot(h_, hT, precision=_HI) * 0.125

    def loop1(i, carry):
        rows = pl.ds(i * RB, RB)
        a1 = a1s[rows, :]
        a2 = a2s[rows, :]
        m1 = jnp.maximum(jnp.max(a1, axis=-1, keepdims=True), 0.0)
        m2 = jnp.maximum(jnp.max(a2, axis=-1, keepdims=True), 0.0)
        p1 = jnp.exp(jnp.maximum(a1, 0.0) - m1)
        p2 = jnp.exp(jnp.maximum(a2, 0.0) - m2)
        z1 = jnp.sum(p1, axis=-1, keepdims=True)
        z2 = jnp.sum(p2, axis=-1, keepdims=True)
        sgn = cwa_ref[rows, :] * (p1 * z2) + cw_ref[rows, :] * (p2 * z1)
        g = jnp.where(sgn > 0, 1.0, 0.0).astype(bf)
        gts[rows, :] = g
        eqs[rows, :] = one_b - g
        zms[rows, 0:1] = z1
        zms[rows, 1:2] = z2
        return carry
    lax.fori_loop(0, N // RB, loop1, 0)

    ones_col = jnp.full((N, 8), one_b, dtype=bf)
    cnt_pos = jnp.dot(gts[...], ones_col,
                      preferred_element_type=f32)[:, :1]

    def _search(_):
        z1 = zms[:, 0:1]
        z2 = zms[:, 1:2]
        m1 = jnp.maximum(jnp.max(a1s[...], axis=-1, keepdims=True), 0.0)
        m2 = jnp.maximum(jnp.max(a2s[...], axis=-1, keepdims=True), 0.0)
        adj1 = jnp.exp(jnp.maximum(a1s[...], 0.0) - m1) / z1
        adj2 = jnp.exp(jnp.maximum(a2s[...], 0.0) - m2) / z2
        s = jnp.maximum(cwa_ref[...] * adj1 + cw_ref[...] * adj2, 0.0)
        s_i = lax.bitcast_convert_type(s, jnp.int32)

        def body(i, thr):
            cand = thr | lax.shift_left(jnp.int32(1), jnp.int32(30) - i)
            cnt = jnp.sum((s_i >= cand).astype(jnp.int32), axis=-1,
                          keepdims=True)
            return jnp.where(cnt >= K, cand, thr)
        thr = lax.fori_loop(0, 31, body, jnp.zeros((N, 1), jnp.int32))
        gtb = jnp.where(s_i > thr, 1.0, 0.0).astype(bf)
        eqb = jnp.where(s_i == thr, 1.0, 0.0).astype(bf)
        gts[...] = gtb
        eqs[...] = eqb
        cnt_gt = jnp.dot(gtb, ones_col, preferred_element_type=f32)[:, :1]
        return jnp.float32(K) - cnt_gt

    def _nosearch(_):
        return jnp.float32(K) - cnt_pos

    budget = lax.cond(jnp.max(cnt_pos) > K, _search, _nosearch, None)
    zms[:, 2:3] = budget

    ranks = a1s
    ranks[...] = jnp.dot(eqs[...], tri_ref[...], preferred_element_type=f32)

    zms[:, 3:4] = jnp.dot(h_, a_ref[:C, :], precision=_HI)
    wh2 = jnp.dot(aP_ref[1:2, :], hT, precision=_HI)

    def loop2(i, carry):
        rows = pl.ds(i * RB, RB)
        tie = jnp.where(ranks[rows, :] <= zms[rows, 2:3],
                        1.0, 0.0).astype(bf)
        keep = gts[rows, :] + eqs[rows, :] * tie
        e = zms[rows, 3:4] + wh2
        att = jnp.maximum(e, 0.01 * e).astype(bf)
        att = jnp.where(keep > jnp.bfloat16(0.0), att, jnp.bfloat16(-1e12))
        p = jnp.exp(att.astype(f32))
        z = jnp.sum(p, axis=-1, keepdims=True)
        adjs[rows, :] = (p * (1.0 / z)).astype(bf)
        return carry
    lax.fori_loop(0, N // RB, loop2, 0)

    adjb = adjs[...]
    xflat = x1.reshape(C * T, N)
    y1 = jnp.dot(xflat, adjb, preferred_element_type=f32).astype(bf)
    y2 = jnp.dot(y1, adjb, preferred_element_type=f32).astype(bf)

    o = (jnp.dot(Wc2_ref[:, :C].astype(bf), y1.reshape(C, T * N),
                 preferred_element_type=f32)
         + jnp.dot(Wc2_ref[:, C:].astype(bf), y2.reshape(C, T * N),
                   preferred_element_type=f32)
         + b2_ref[...])
    out_ref[0] = (o.reshape(C * T, N) * emb_ref[0, 0] + xt)


def kernel(x, Wc1, b1, Wg, a, memory, cwa, cw, Wc2, b2, emb):
    B, C, N, T = _B, _C, _N, _T
    xt = jnp.transpose(x, (0, 1, 3, 2)).reshape(B, C * T, N)
    aP = jnp.concatenate([a.reshape(2, C), jnp.zeros((6, C), a.dtype)], 0)
    tri = (jnp.arange(N)[:, None] <= jnp.arange(N)[None, :]).astype(
        jnp.bfloat16)
    full = lambda shp: pl.BlockSpec(shp, lambda b: (0,) * len(shp))
    out = pl.pallas_call(
        _dgcn_kernel,
        grid=(B,),
        in_specs=[
            pl.BlockSpec((1, C * T, N), lambda b: (b, 0, 0)),
            full((C, C)), full((C, 1)), full((C, C)), full((2 * C, 1)),
            full((8, C)), full((C, N)), full((N, N)), full((N, N)),
            full((C, 2 * C)), full((C, 1)), full((1, 1)), full((N, N)),
        ],
        out_specs=pl.BlockSpec((1, C * T, N), lambda b: (b, 0, 0)),
        out_shape=jax.ShapeDtypeStruct((B, C * T, N), jnp.float32),
        scratch_shapes=[
            pltpu.VMEM((N, N), jnp.float32),
            pltpu.VMEM((N, N), jnp.float32),
            pltpu.VMEM((N, N), jnp.bfloat16),
            pltpu.VMEM((N, N), jnp.bfloat16),
            pltpu.VMEM((N, N), jnp.bfloat16),
            pltpu.VMEM((N, 8), jnp.float32),
        ],
        compiler_params=pltpu.CompilerParams(
            dimension_semantics=("arbitrary",)),
    )(xt, Wc1, b1.reshape(C, 1), Wg.T, a, aP, memory.T, cwa, cw,
      Wc2, b2.reshape(C, 1), emb.reshape(1, 1), tri)
    return jnp.transpose(out.reshape(B, C, T, N), (0, 1, 3, 2))


# speedup vs baseline: 1.3916x; 1.3916x over previous
import jax
import jax.numpy as jnp
from jax import lax
from jax.experimental import pallas as pl
from jax.experimental.pallas import tpu as pltpu

_B, _C, _N, _T = 4, 64, 1024, 12
_K = int(_N * 0.8)
_HI = lax.Precision.HIGHEST


def _dgcn_kernel(xt_ref, Wc1_ref, b1_ref, WgT_ref, a_ref, aP_ref, memT_ref,
                 cwa_ref, cw_ref, Wc2_ref, b2_ref, emb_ref, out_ref):
    C, N, T, K = _C, _N, _T, _K
    bf = jnp.bfloat16
    f32 = jnp.float32
    xt = xt_ref[0]
    x2 = xt.reshape(C, T * N)

    x1 = jnp.dot(Wc1_ref[...].astype(bf), x2.astype(bf),
                 preferred_element_type=f32)
    x1 = (x1 + b1_ref[...]).astype(bf)

    xs = (jnp.dot(Wc1_ref[...], xt.reshape(C, T, N).sum(axis=1),
                  precision=_HI) + jnp.float32(T) * b1_ref[...])
    hT = jnp.dot(WgT_ref[...], xs, precision=_HI)
    h_ = hT.T
    h8 = h_ * 0.125

    wh1 = jnp.dot(h_, a_ref[:C, :], precision=_HI)
    wh2 = jnp.dot(aP_ref[1:2, :], hT, precision=_HI)
    e = wh1 + wh2
    att = jnp.maximum(e, 0.01 * e).astype(bf)

    a1r = jnp.dot(h8, memT_ref[...], precision=_HI)
    a2r = jnp.dot(h8, hT, precision=_HI)

    def _rowsoftmax(z):
        z = jnp.maximum(z, 0.0)
        m = jnp.max(z, axis=-1, keepdims=True)
        p = jnp.exp(z - m)
        return p / jnp.sum(p, axis=-1, keepdims=True)

    adj1 = _rowsoftmax(a1r)
    adj2 = _rowsoftmax(a2r)
    s = jnp.maximum(cwa_ref[...] * adj1 + cw_ref[...] * adj2, 0.0)

    s_i = lax.bitcast_convert_type(s, jnp.int32)
    one_b = jnp.bfloat16(1.0)
    ones_col = jnp.full((N, 8), one_b, dtype=bf)
    gt0 = jnp.where(s_i > 0, 1.0, 0.0).astype(bf)
    cnt_pos = jnp.dot(gt0, ones_col,
                      preferred_element_type=f32)[:, :1]

    def _search(_):
        def body(i, thr):
            cand = thr | lax.shift_left(jnp.int32(1), jnp.int32(30) - i)
            cnt = jnp.sum((s_i >= cand).astype(jnp.int32), axis=-1,
                          keepdims=True)
            return jnp.where(cnt >= K, cand, thr)
        thr = lax.fori_loop(0, 31, body, jnp.zeros((N, 1), jnp.int32))
        gtb = jnp.where(s_i > thr, 1.0, 0.0).astype(bf)
        eqb = jnp.where(s_i == thr, 1.0, 0.0).astype(bf)
        cnt_gt = jnp.dot(gtb, ones_col, preferred_element_type=f32)[:, :1]
        return gtb, eqb, cnt_gt

    def _nosearch(_):
        return gt0, one_b - gt0, cnt_pos

    gtb, eqb, cnt_gt = lax.cond(jnp.max(cnt_pos) > K, _search, _nosearch,
                                None)

    ii = lax.broadcasted_iota(jnp.int32, (N, N), 0)
    jj = lax.broadcasted_iota(jnp.int32, (N, N), 1)
    tri = jnp.where(ii <= jj, 1.0, 0.0).astype(bf)
    tie_rank = jnp.dot(eqb, tri, preferred_element_type=f32)

    keep = gtb + eqb * jnp.where(tie_rank <= (K - cnt_gt), 1.0, 0.0).astype(bf)
    att = jnp.where(keep > jnp.bfloat16(0.0), att, jnp.bfloat16(-1e12))
    p = jnp.exp(att)
    z = jnp.dot(p, ones_col, preferred_element_type=f32)[:, :1]
    adjb = p * (1.0 / z).astype(bf)

    xflat = x1.reshape(C * T, N)
    y1 = jnp.dot(xflat, adjb, preferred_element_type=f32).astype(bf)
    y2 = jnp.dot(y1, adjb, preferred_element_type=f32).astype(bf)

    o = (jnp.dot(Wc2_ref[:, :C].astype(bf), y1.reshape(C, T * N),
                 preferred_element_type=f32)
         + jnp.dot(Wc2_ref[:, C:].astype(bf), y2.reshape(C, T * N),
                   preferred_element_type=f32)
         + b2_ref[...])
    out_ref[0] = (o.reshape(C * T, N) * emb_ref[0, 0] + xt)


def kernel(x, Wc1, b1, Wg, a, memory, cwa, cw, Wc2, b2, emb):
    B, C, N, T = _B, _C, _N, _T
    xt = jnp.transpose(x, (0, 1, 3, 2)).reshape(B, C * T, N)
    aP = jnp.concatenate([a.reshape(2, C), jnp.zeros((6, C), a.dtype)], 0)
    full = lambda shp: pl.BlockSpec(shp, lambda b: (0,) * len(shp))
    out = pl.pallas_call(
        _dgcn_kernel,
        grid=(B,),
        in_specs=[
            pl.BlockSpec((1, C * T, N), lambda b: (b, 0, 0)),
            full((C, C)), full((C, 1)), full((C, C)), full((2 * C, 1)),
            full((8, C)), full((C, N)), full((N, N)), full((N, N)),
            full((C, 2 * C)), full((C, 1)), full((1, 1)),
        ],
        out_specs=pl.BlockSpec((1, C * T, N), lambda b: (b, 0, 0)),
        out_shape=jax.ShapeDtypeStruct((B, C * T, N), jnp.float32),
        compiler_params=pltpu.CompilerParams(
            dimension_semantics=("arbitrary",)),
    )(xt, Wc1, b1.reshape(C, 1), Wg.T, a, aP, memory.T, cwa, cw,
      Wc2, b2.reshape(C, 1), emb.reshape(1, 1))
    return jnp.transpose(out.reshape(B, C, T, N), (0, 1, 3, 2))


# speedup vs baseline: 1.4705x; 1.0567x over previous
import jax
import jax.numpy as jnp
from jax import lax
from jax.experimental import pallas as pl
from jax.experimental.pallas import tpu as pltpu

_B, _C, _N, _T = 4, 64, 1024, 12
_K = int(_N * 0.8)
_HI = lax.Precision.HIGHEST


def _dgcn_kernel(xt_ref, Wc1_ref, b1_ref, WgT_ref, a_ref, aP_ref, memT_ref,
                 cwa_ref, cw_ref, Wc2_ref, b2_ref, emb_ref, out_ref):
    C, N, T, K = _C, _N, _T, _K
    xt = xt_ref[0]
    x2 = xt.reshape(C, T * N)

    bf = jnp.bfloat16
    x1 = jnp.dot(Wc1_ref[...].astype(bf), x2.astype(bf),
                 preferred_element_type=jnp.float32)
    x1 = x1 + b1_ref[...]

    xs = (jnp.dot(Wc1_ref[...], xt.reshape(C, T, N).sum(axis=1),
                  precision=_HI) + jnp.float32(T) * b1_ref[...])
    hT = jnp.dot(WgT_ref[...], xs, precision=_HI)
    h_ = hT.T

    a1r = jnp.dot(h_, memT_ref[...], precision=_HI) * 0.125
    a2r = jnp.dot(h_, hT, precision=_HI) * 0.125

    def _rowsoftmax(z):
        z = jnp.maximum(z, 0.0)
        m = jnp.max(z, axis=-1, keepdims=True)
        p = jnp.exp(z - m)
        return p / jnp.sum(p, axis=-1, keepdims=True)

    adj1 = _rowsoftmax(a1r)
    adj2 = _rowsoftmax(a2r)
    s = jnp.maximum(cwa_ref[...] * adj1 + cw_ref[...] * adj2, 0.0)

    s_i = lax.bitcast_convert_type(s, jnp.int32)
    pos = (s_i > 0).astype(jnp.int32)
    cnt_pos = jnp.sum(pos, axis=-1, keepdims=True)

    def _search(_):
        def body(i, thr):
            cand = thr | lax.shift_left(jnp.int32(1), jnp.int32(30) - i)
            cnt = jnp.sum((s_i >= cand).astype(jnp.int32), axis=-1,
                          keepdims=True)
            return jnp.where(cnt >= K, cand, thr)
        return lax.fori_loop(0, 31, body, jnp.zeros((N, 1), jnp.int32))

    thr = lax.cond(jnp.max(cnt_pos) > K, _search,
                   lambda _: jnp.zeros((N, 1), jnp.int32), None)

    gt = s_i > thr
    eq = s_i == thr
    cnt_gt = jnp.sum(gt.astype(jnp.int32), axis=-1, keepdims=True)
    ii = lax.broadcasted_iota(jnp.int32, (N, N), 0)
    jj = lax.broadcasted_iota(jnp.int32, (N, N), 1)
    tri = (ii <= jj).astype(bf)
    tie_rank = jnp.dot(eq.astype(bf), tri, preferred_element_type=jnp.float32)
    mask = gt | (eq & (tie_rank <= (K - cnt_gt).astype(jnp.float32)))

    wh1 = jnp.dot(h_, a_ref[:C, :], precision=_HI)
    wh2 = jnp.dot(aP_ref[1:2, :], hT, precision=_HI)
    e = wh1 + wh2
    att = jnp.where(e >= 0, e, 0.01 * e)
    att = jnp.where(mask, att, jnp.float32(-1e12))
    m = jnp.max(att, axis=-1, keepdims=True)
    p = jnp.exp(att - m)
    adj = p / jnp.sum(p, axis=-1, keepdims=True)

    f32 = jnp.float32
    adjb = adj.astype(bf)
    xflat = x1.reshape(C * T, N).astype(bf)
    y1 = jnp.dot(xflat, adjb, preferred_element_type=f32)
    y2 = jnp.dot(y1.astype(bf), adjb, preferred_element_type=f32)

    o = (jnp.dot(Wc2_ref[:, :C].astype(bf), y1.reshape(C, T * N).astype(bf),
                 preferred_element_type=f32)
         + jnp.dot(Wc2_ref[:, C:].astype(bf), y2.reshape(C, T * N).astype(bf),
                   preferred_element_type=f32)
         + b2_ref[...])
    out_ref[0] = (o.reshape(C * T, N) * emb_ref[0, 0] + xt)


def kernel(x, Wc1, b1, Wg, a, memory, cwa, cw, Wc2, b2, emb):
    B, C, N, T = _B, _C, _N, _T
    xt = jnp.transpose(x, (0, 1, 3, 2)).reshape(B, C * T, N)
    aP = jnp.concatenate([a.reshape(2, C), jnp.zeros((6, C), a.dtype)], 0)
    full = lambda shp: pl.BlockSpec(shp, lambda b: (0,) * len(shp))
    out = pl.pallas_call(
        _dgcn_kernel,
        grid=(B,),
        in_specs=[
            pl.BlockSpec((1, C * T, N), lambda b: (b, 0, 0)),
            full((C, C)), full((C, 1)), full((C, C)), full((2 * C, 1)),
            full((8, C)), full((C, N)), full((N, N)), full((N, N)),
            full((C, 2 * C)), full((C, 1)), full((1, 1)),
        ],
        out_specs=pl.BlockSpec((1, C * T, N), lambda b: (b, 0, 0)),
        out_shape=jax.ShapeDtypeStruct((B, C * T, N), jnp.float32),
        compiler_params=pltpu.CompilerParams(
            dimension_semantics=("arbitrary",)),
    )(xt, Wc1, b1.reshape(C, 1), Wg.T, a, aP, memory.T, cwa, cw,
      Wc2, b2.reshape(C, 1), emb.reshape(1, 1))
    return jnp.transpose(out.reshape(B, C, T, N), (0, 1, 3, 2))


# speedup vs baseline: 1.4732x; 1.0019x over previous
import jax
import jax.numpy as jnp
from jax import lax
from jax.experimental import pallas as pl
from jax.experimental.pallas import tpu as pltpu

_B, _C, _N, _T = 4, 64, 1024, 12
_K = int(_N * 0.8)
_HI = lax.Precision.HIGHEST


def _dgcn_kernel(xt_ref, Wc1_ref, b1_ref, WgT_ref, a_ref, aP_ref, memT_ref,
                 cwa_ref, cw_ref, Wc2_ref, b2_ref, emb_ref, out_ref):
    C, N, T, K = _C, _N, _T, _K
    xt = xt_ref[0]
    x2 = xt.reshape(C, T * N)

    bf = jnp.bfloat16
    x1 = jnp.dot(Wc1_ref[...].astype(bf), x2.astype(bf),
                 preferred_element_type=jnp.float32)
    x1 = x1 + b1_ref[...]

    xs = (jnp.dot(Wc1_ref[...], xt.reshape(C, T, N).sum(axis=1),
                  precision=_HI) + jnp.float32(T) * b1_ref[...])
    hT = jnp.dot(WgT_ref[...], xs, precision=_HI)
    h_ = hT.T
    h8 = h_ * 0.125

    a1r = jnp.dot(h8, memT_ref[...], precision=_HI)
    a2r = jnp.dot(h8, hT, precision=_HI)

    def _rowsoftmax(z):
        z = jnp.maximum(z, 0.0)
        m = jnp.max(z, axis=-1, keepdims=True)
        p = jnp.exp(z - m)
        return p / jnp.sum(p, axis=-1, keepdims=True)

    adj1 = _rowsoftmax(a1r)
    adj2 = _rowsoftmax(a2r)
    s = jnp.maximum(cwa_ref[...] * adj1 + cw_ref[...] * adj2, 0.0)

    s_i = lax.bitcast_convert_type(s, jnp.int32)
    pos = (s_i > 0).astype(jnp.int32)
    cnt_pos = jnp.sum(pos, axis=-1, keepdims=True)

    def _search(_):
        def body(i, thr):
            cand = thr | lax.shift_left(jnp.int32(1), jnp.int32(30) - i)
            cnt = jnp.sum((s_i >= cand).astype(jnp.int32), axis=-1,
                          keepdims=True)
            return jnp.where(cnt >= K, cand, thr)
        return lax.fori_loop(0, 31, body, jnp.zeros((N, 1), jnp.int32))

    thr = lax.cond(jnp.max(cnt_pos) > K, _search,
                   lambda _: jnp.zeros((N, 1), jnp.int32), None)

    gt = s_i > thr
    eq = s_i == thr
    cnt_gt = jnp.sum(gt.astype(jnp.int32), axis=-1, keepdims=True)
    ii = lax.broadcasted_iota(jnp.int32, (N, N), 0)
    jj = lax.broadcasted_iota(jnp.int32, (N, N), 1)
    tri = (ii <= jj).astype(bf)
    tie_rank = jnp.dot(eq.astype(bf), tri, preferred_element_type=jnp.float32)
    mask = gt | (eq & (tie_rank <= (K - cnt_gt).astype(jnp.float32)))

    wh1 = jnp.dot(h_, a_ref[:C, :], precision=_HI)
    wh2 = jnp.dot(aP_ref[1:2, :], hT, precision=_HI)
    e = wh1 + wh2
    att = jnp.where(e >= 0, e, 0.01 * e)
    att = jnp.where(mask, att, jnp.float32(-1e12))
    m = jnp.max(att, axis=-1, keepdims=True)
    p = jnp.exp(att - m)
    adj = p / jnp.sum(p, axis=-1, keepdims=True)

    f32 = jnp.float32
    adjb = adj.astype(bf)
    xflat = x1.reshape(C * T, N).astype(bf)
    y1 = jnp.dot(xflat, adjb, preferred_element_type=f32)
    y2 = jnp.dot(y1.astype(bf), adjb, preferred_element_type=f32)

    o = (jnp.dot(Wc2_ref[:, :C].astype(bf), y1.reshape(C, T * N).astype(bf),
                 preferred_element_type=f32)
         + jnp.dot(Wc2_ref[:, C:].astype(bf), y2.reshape(C, T * N).astype(bf),
                   preferred_element_type=f32)
         + b2_ref[...])
    out_ref[0] = (o.reshape(C * T, N) * emb_ref[0, 0] + xt)


def kernel(x, Wc1, b1, Wg, a, memory, cwa, cw, Wc2, b2, emb):
    B, C, N, T = _B, _C, _N, _T
    xt = jnp.transpose(x, (0, 1, 3, 2)).reshape(B, C * T, N)
    aP = jnp.concatenate([a.reshape(2, C), jnp.zeros((6, C), a.dtype)], 0)
    full = lambda shp: pl.BlockSpec(shp, lambda b: (0,) * len(shp))
    out = pl.pallas_call(
        _dgcn_kernel,
        grid=(B,),
        in_specs=[
            pl.BlockSpec((1, C * T, N), lambda b: (b, 0, 0)),
            full((C, C)), full((C, 1)), full((C, C)), full((2 * C, 1)),
            full((8, C)), full((C, N)), full((N, N)), full((N, N)),
            full((C, 2 * C)), full((C, 1)), full((1, 1)),
        ],
        out_specs=pl.BlockSpec((1, C * T, N), lambda b: (b, 0, 0)),
        out_shape=jax.ShapeDtypeStruct((B, C * T, N), jnp.float32),
        compiler_params=pltpu.CompilerParams(
            dimension_semantics=("arbitrary",)),
    )(xt, Wc1, b1.reshape(C, 1), Wg.T, a, aP, memory.T, cwa, cw,
      Wc2, b2.reshape(C, 1), emb.reshape(1, 1))
    return jnp.transpose(out.reshape(B, C, T, N), (0, 1, 3, 2))


# speedup vs baseline: 1.4773x; 1.0028x over previous
import jax
import jax.numpy as jnp
from jax import lax
from jax.experimental import pallas as pl
from jax.experimental.pallas import tpu as pltpu

_B, _C, _N, _T = 4, 64, 1024, 12
_K = int(_N * 0.8)
_HI = lax.Precision.HIGHEST


def _dgcn_kernel(xt_ref, Wc1_ref, b1_ref, WgT_ref, a_ref, aP_ref, memT_ref,
                 cwa_ref, cw_ref, Wc2_ref, b2_ref, emb_ref, out_ref):
    C, N, T, K = _C, _N, _T, _K
    xt = xt_ref[0]
    x2 = xt.reshape(C, T * N)

    bf = jnp.bfloat16
    x1 = jnp.dot(Wc1_ref[...].astype(bf), x2.astype(bf),
                 preferred_element_type=jnp.float32)
    x1 = x1 + b1_ref[...]

    xs = (jnp.dot(Wc1_ref[...], xt.reshape(C, T, N).sum(axis=1),
                  precision=_HI) + jnp.float32(T) * b1_ref[...])
    hT = jnp.dot(WgT_ref[...], xs, precision=_HI)
    h_ = hT.T
    h8 = h_ * 0.125

    a1r = jnp.dot(h8, memT_ref[...], precision=_HI)
    a2r = jnp.dot(h8, hT, precision=_HI)

    def _rowsoftmax(z):
        z = jnp.maximum(z, 0.0)
        m = jnp.max(z, axis=-1, keepdims=True)
        p = jnp.exp(z - m)
        return p / jnp.sum(p, axis=-1, keepdims=True)

    adj1 = _rowsoftmax(a1r)
    adj2 = _rowsoftmax(a2r)
    s = jnp.maximum(cwa_ref[...] * adj1 + cw_ref[...] * adj2, 0.0)

    s_i = lax.bitcast_convert_type(s, jnp.int32)
    pos = (s_i > 0).astype(jnp.int32)
    cnt_pos = jnp.sum(pos, axis=-1, keepdims=True)

    def _search(_):
        def body(i, thr):
            cand = thr | lax.shift_left(jnp.int32(1), jnp.int32(30) - i)
            cnt = jnp.sum((s_i >= cand).astype(jnp.int32), axis=-1,
                          keepdims=True)
            return jnp.where(cnt >= K, cand, thr)
        thr = lax.fori_loop(0, 31, body, jnp.zeros((N, 1), jnp.int32))
        gtc = jnp.sum((s_i > thr).astype(jnp.int32), axis=-1, keepdims=True)
        return thr, gtc

    def _nosearch(_):
        return jnp.zeros((N, 1), jnp.int32), cnt_pos

    thr, cnt_gt = lax.cond(jnp.max(cnt_pos) > K, _search, _nosearch, None)

    gt = s_i > thr
    eq = s_i == thr
    ii = lax.broadcasted_iota(jnp.int32, (N, N), 0)
    jj = lax.broadcasted_iota(jnp.int32, (N, N), 1)
    tri = (ii <= jj).astype(bf)
    tie_rank = jnp.dot(eq.astype(bf), tri, preferred_element_type=jnp.float32)
    mask = gt | (eq & (tie_rank <= (K - cnt_gt).astype(jnp.float32)))

    wh1 = jnp.dot(h_, a_ref[:C, :], precision=_HI)
    wh2 = jnp.dot(aP_ref[1:2, :], hT, precision=_HI)
    e = wh1 + wh2
    att = jnp.where(e >= 0, e, 0.01 * e)
    att = jnp.where(mask, att, jnp.float32(-1e12))
    m = jnp.max(att, axis=-1, keepdims=True)
    p = jnp.exp(att - m)
    adj = p / jnp.sum(p, axis=-1, keepdims=True)

    f32 = jnp.float32
    adjb = adj.astype(bf)
    xflat = x1.reshape(C * T, N).astype(bf)
    y1 = jnp.dot(xflat, adjb, preferred_element_type=f32)
    y2 = jnp.dot(y1.astype(bf), adjb, preferred_element_type=f32)

    o = (jnp.dot(Wc2_ref[:, :C].astype(bf), y1.reshape(C, T * N).astype(bf),
                 preferred_element_type=f32)
         + jnp.dot(Wc2_ref[:, C:].astype(bf), y2.reshape(C, T * N).astype(bf),
                   preferred_element_type=f32)
         + b2_ref[...])
    out_ref[0] = (o.reshape(C * T, N) * emb_ref[0, 0] + xt)


def kernel(x, Wc1, b1, Wg, a, memory, cwa, cw, Wc2, b2, emb):
    B, C, N, T = _B, _C, _N, _T
    xt = jnp.transpose(x, (0, 1, 3, 2)).reshape(B, C * T, N)
    aP = jnp.concatenate([a.reshape(2, C), jnp.zeros((6, C), a.dtype)], 0)
    full = lambda shp: pl.BlockSpec(shp, lambda b: (0,) * len(shp))
    out = pl.pallas_call(
        _dgcn_kernel,
        grid=(B,),
        in_specs=[
            pl.BlockSpec((1, C * T, N), lambda b: (b, 0, 0)),
            full((C, C)), full((C, 1)), full((C, C)), full((2 * C, 1)),
            full((8, C)), full((C, N)), full((N, N)), full((N, N)),
            full((C, 2 * C)), full((C, 1)), full((1, 1)),
        ],
        out_specs=pl.BlockSpec((1, C * T, N), lambda b: (b, 0, 0)),
        out_shape=jax.ShapeDtypeStruct((B, C * T, N), jnp.float32),
        compiler_params=pltpu.CompilerParams(
            dimension_semantics=("arbitrary",)),
    )(xt, Wc1, b1.reshape(C, 1), Wg.T, a, aP, memory.T, cwa, cw,
      Wc2, b2.reshape(C, 1), emb.reshape(1, 1))
    return jnp.transpose(out.reshape(B, C, T, N), (0, 1, 3, 2))
